# baseline (speedup 1.0000x reference)
import jax
import jax.numpy as jnp
from jax.experimental import pallas as pl
from jax.experimental.pallas import tpu as pltpu

NUM_GRAPHS = 128
HIDDEN = 256
BLOCK = 512


def _pool_kernel(firsts, lasts, x_ref, seg_ref, segr_ref, wt_ref, b_ref,
                 o_ref, mx_ref, sm_ref, ct_ref):
    i = pl.program_id(0)
    nb = pl.num_programs(0)

    @pl.when(i == 0)
    def _():
        mx_ref[...] = jnp.full_like(mx_ref, -jnp.inf)
        sm_ref[...] = jnp.zeros_like(sm_ref)
        ct_ref[...] = jnp.zeros_like(ct_ref)

    x = x_ref[...]
    seg = seg_ref[...]
    first = firsts[i]
    last = lasts[i]

    seg_row = segr_ref[0]
    gids = jax.lax.broadcasted_iota(jnp.int32, (NUM_GRAPHS, BLOCK), 0)
    onehot = jnp.where(gids == seg_row, 1.0, 0.0)
    bsum = jax.lax.dot_general(
        onehot, x, (((1,), (0,)), ((), ())),
        preferred_element_type=jnp.float32,
        precision=jax.lax.Precision.HIGHEST)
    sm_ref[...] += bsum
    bcnt = jnp.sum(onehot, axis=1, keepdims=True)
    ct_ref[...] += jnp.broadcast_to(bcnt, (NUM_GRAPHS, HIDDEN))

    def body(s, carry):
        m = seg == s
        xm = jnp.where(m, x, -jnp.inf)
        bmax = jnp.max(xm, axis=0, keepdims=True)
        mx_ref[pl.ds(s, 1), :] = jnp.maximum(mx_ref[pl.ds(s, 1), :], bmax)
        return carry

    jax.lax.fori_loop(first, last + 1, body, 0)

    @pl.when(i == nb - 1)
    def _():
        mean = sm_ref[...] / jnp.maximum(ct_ref[...], 1.0)
        comb = jnp.concatenate([mx_ref[...], mean], axis=1)
        o_ref[...] = jax.lax.dot_general(
            comb, wt_ref[...], (((1,), (0,)), ((), ())),
            preferred_element_type=jnp.float32) + b_ref[...]


@jax.jit
def kernel(x, batch, W, b):
    n, h = x.shape
    batch = batch.astype(jnp.int32)
    nb = pl.cdiv(n, BLOCK)
    npad = nb * BLOCK
    x = jnp.pad(x, ((0, npad - n), (0, 0)))
    segp = jnp.pad(batch, (0, npad - n), constant_values=NUM_GRAPHS)
    firsts = segp[::BLOCK]
    lasts = jnp.minimum(segp[BLOCK - 1::BLOCK], NUM_GRAPHS - 1)
    seg2d = segp.reshape(npad, 1)
    seg3d = segp.reshape(nb, 1, BLOCK)
    wt = W.T
    b2 = b.reshape(1, h)

    out = pl.pallas_call(
        _pool_kernel,
        grid_spec=pltpu.PrefetchScalarGridSpec(
            num_scalar_prefetch=2,
            grid=(nb,),
            in_specs=[
                pl.BlockSpec((BLOCK, h), lambda i, *_: (i, 0)),
                pl.BlockSpec((BLOCK, 1), lambda i, *_: (i, 0)),
                pl.BlockSpec((1, 1, BLOCK), lambda i, *_: (i, 0, 0)),
                pl.BlockSpec((2 * h, h), lambda i, *_: (0, 0)),
                pl.BlockSpec((1, h), lambda i, *_: (0, 0)),
            ],
            out_specs=pl.BlockSpec((NUM_GRAPHS, h), lambda i, *_: (0, 0)),
            scratch_shapes=[
                pltpu.VMEM((NUM_GRAPHS, h), jnp.float32),
                pltpu.VMEM((NUM_GRAPHS, h), jnp.float32),
                pltpu.VMEM((NUM_GRAPHS, h), jnp.float32),
            ],
        ),
        out_shape=jax.ShapeDtypeStruct((NUM_GRAPHS, h), jnp.float32),
    )(firsts, lasts, x, seg2d, seg3d, wt, b2)
    return out
